# baseline (speedup 1.0000x reference)
import dataclasses
import functools

import jax
import jax.numpy as jnp
import numpy as np
from jax import lax
from jax.experimental import pallas as pl
from jax.experimental.pallas import tpu as pltpu
from jax.experimental.pallas import tpu_sc as plsc

N = 10000
NPAD = 10240
E = 160000
D = 256
DH = 128
NS = 16
NC = 2
PT = 10240
EPAD = NS * PT
CH = 80
NBLK = 8
BCH = 16
RPT = NPAD // NS
WCH = 80

_PERM = np.empty((D,), np.int32)
for _h in (0, DH):
    for _g in range(DH // 32):
        for _i in range(16):
            _PERM[_h + 32 * _g + 2 * _i] = _h + 32 * _g + _i
            _PERM[_h + 32 * _g + 2 * _i + 1] = _h + 32 * _g + 16 + _i


def _matmul_body(x_ref, w_ref, o_ref):
    o_ref[...] = jnp.dot(x_ref[...], w_ref[...],
                         preferred_element_type=jnp.float32
                         ).astype(jnp.bfloat16)


def _compute_hsplit(x, Wp):
    return pl.pallas_call(
        _matmul_body,
        grid=(10, 2),
        in_specs=[
            pl.BlockSpec((1000, D), lambda i, j: (i, 0)),
            pl.BlockSpec((D, DH), lambda i, j: (0, j)),
        ],
        out_specs=pl.BlockSpec((1000, DH), lambda i, j: (j * 10 + i, 0)),
        out_shape=jax.ShapeDtypeStruct((NC * N, DH), jnp.bfloat16),
    )(x, Wp)


_vector_mesh = plsc.VectorSubcoreMesh(core_axis_name="c", subcore_axis_name="s")

_sc_compiler_params = pltpu.CompilerParams()
if "needs_layout_passes" in pltpu.CompilerParams.__dataclass_fields__:
    _sc_compiler_params = dataclasses.replace(
        _sc_compiler_params, needs_layout_passes=False)
if "use_tc_tiling_on_sc" in pltpu.CompilerParams.__dataclass_fields__:
    _sc_compiler_params = dataclasses.replace(
        _sc_compiler_params, use_tc_tiling_on_sc=False)


@functools.partial(
    pl.kernel,
    out_type=jax.ShapeDtypeStruct((N, D), jnp.float32),
    mesh=_vector_mesh,
    compiler_params=_sc_compiler_params,
    scratch_types=[
        pltpu.VMEM_SHARED((NPAD, DH), jnp.float32),
        pltpu.VMEM((BCH, CH), jnp.int32),
        pltpu.VMEM((BCH, CH), jnp.int32),
        pltpu.VMEM((BCH, CH), jnp.int32),
        pltpu.VMEM((BCH, CH), jnp.int32),
        pltpu.VMEM((BCH, CH), jnp.float32),
        pltpu.VMEM((BCH, CH), jnp.float32),
        pltpu.VMEM((CH, DH // 2), jnp.int32),
        pltpu.VMEM((CH, DH // 2), jnp.int32),
        pltpu.VMEM((CH, DH // 2), jnp.int32),
        pltpu.VMEM((CH, DH // 2), jnp.int32),
        pltpu.VMEM((WCH, DH), jnp.float32),
        pltpu.VMEM((WCH, DH), jnp.float32),
        pltpu.SemaphoreType.DMA,
        pltpu.SemaphoreType.DMA,
        pltpu.SemaphoreType.DMA,
        pltpu.SemaphoreType.DMA,
        pltpu.SemaphoreType.DMA,
        pltpu.SemaphoreType.DMA,
    ],
)
def _sc_aggregate(h_hbm, row_hbm, col_hbm, adj_hbm, z_hbm, out_hbm,
                  accum, rowA, rowB, colA, colB, valA, valB,
                  gbuf0, gbuf1, gbuf2, gbuf3, msg0, msg1,
                  gsem0, gsem1, gsem2, gsem3, ssem0, ssem1):
    c = lax.axis_index("c")
    s = lax.axis_index("s")
    r0 = s * RPT
    gbufs = (gbuf0, gbuf1, gbuf2, gbuf3)
    gsems = (gsem0, gsem1, gsem2, gsem3)
    msgs = (msg0, msg1)
    ssems = (ssem0, ssem1)
    rows = (rowA, rowB)
    cols = (colA, colB)
    vals = (valA, valB)
    NPAIR = NBLK // 2

    pltpu.sync_copy(z_hbm.at[pl.ds(r0, RPT)], accum.at[pl.ds(r0, RPT)])
    plsc.subcore_barrier()

    def load_idx(b, p):
        pltpu.sync_copy(row_hbm.at[s, b], rows[p])
        pltpu.sync_copy(col_hbm.at[c, s, b], cols[p])
        pltpu.sync_copy(adj_hbm.at[s, b], vals[p])

    def scale(gbuf, msg, val, j):
        @plsc.parallel_loop(0, CH, unroll=2)
        def _row(r):
            v = plsc.load_gather(
                val,
                [jnp.full((16,), j, jnp.int32),
                 jnp.full((16,), r, jnp.int32)])
            for g in range(DH // 32):
                w = gbuf[r, pl.ds(16 * g, 16)]
                ab = plsc.bitcast(w, jnp.bfloat16)
                va, vb = plsc.unpack(ab, format=plsc.PackFormat.INTERLEAVED)
                msg[r, pl.ds(32 * g, 16)] = va * v
                msg[r, pl.ds(32 * g + 16, 16)] = vb * v

    def wait_scatter(p, jj, m):
        pltpu.make_async_copy(msgs[m], accum.at[rows[p].at[jj]],
                              ssems[m]).wait()

    load_idx(0, 0)
    for i in range(4):
        pltpu.async_copy(h_hbm.at[colA.at[i]], gbufs[i], gsems[i])

    @pl.loop(0, NPAIR)
    def _pair(t):
        for p in range(2):
            for jj in range(BCH):
                i = jj % 4
                m = jj % 2
                if jj == BCH - 4:
                    if p == 0:
                        load_idx(2 * t + 1, 1)
                    else:
                        @pl.when(t < NPAIR - 1)
                        def _():
                            load_idx(2 * t + 2, 0)

                pltpu.make_async_copy(h_hbm.at[cols[p].at[jj]], gbufs[i],
                                      gsems[i]).wait()

                if jj >= 2:
                    wait_scatter(p, jj - 2, m)
                elif p == 1:
                    wait_scatter(0, jj + BCH - 2, m)
                else:
                    @pl.when(t > 0)
                    def _():
                        wait_scatter(1, jj + BCH - 2, m)

                scale(gbufs[i], msgs[m], vals[p], jj)
                pltpu.async_copy(msgs[m], accum.at[rows[p].at[jj]],
                                 ssems[m], add=True)

                if jj < BCH - 4:
                    pltpu.async_copy(h_hbm.at[cols[p].at[jj + 4]],
                                     gbufs[i], gsems[i])
                elif p == 0:
                    pltpu.async_copy(h_hbm.at[colB.at[jj - (BCH - 4)]],
                                     gbufs[i], gsems[i])
                else:
                    @pl.when(t < NPAIR - 1)
                    def _():
                        pltpu.async_copy(h_hbm.at[colA.at[jj - (BCH - 4)]],
                                         gbufs[i], gsems[i])

    wait_scatter(1, BCH - 2, 0)
    wait_scatter(1, BCH - 1, 1)

    plsc.subcore_barrier()

    for t in range(RPT // WCH):
        base = r0 + t * WCH

        @pl.when(base < N)
        def _write():
            pltpu.sync_copy(accum.at[pl.ds(base, WCH)], msg0)

            @plsc.parallel_loop(0, WCH, unroll=2)
            def _relu_row(r):
                for k in range(DH // 16):
                    sl = (r, pl.ds(k * 16, 16))
                    msg0[sl] = jnp.maximum(msg0[sl], 0.0)

            pltpu.sync_copy(
                msg0, out_hbm.at[pl.ds(base, WCH), pl.ds(c * DH, DH)])


def kernel(x, edge_index, adj_values, features_nonzero, W):
    row = edge_index[0].astype(jnp.int32)
    col = edge_index[1].astype(jnp.int32)
    pad = EPAD - E
    row_p = jnp.concatenate([row, jnp.full((pad,), N, jnp.int32)])
    col_p = jnp.concatenate([col, jnp.zeros((pad,), jnp.int32)])
    adj_p = jnp.concatenate([adj_values, jnp.zeros((pad,), jnp.float32)])
    hsplit = _compute_hsplit(x, W[:, _PERM])
    h32 = lax.bitcast_convert_type(
        hsplit.reshape(NC * N, DH // 2, 2), jnp.int32)
    row4 = row_p.reshape(NS, NBLK, BCH, CH)
    col5 = jnp.stack([col_p, col_p + N]).reshape(NC, NS, NBLK, BCH, CH)
    adj4 = adj_p.reshape(NS, NBLK, BCH, CH)
    zeros = jnp.zeros((NPAD, DH), jnp.float32)
    return _sc_aggregate(h32, row4, col5, adj4, zeros)


# speedup vs baseline: 1.0060x; 1.0060x over previous
import dataclasses
import functools

import jax
import jax.numpy as jnp
import numpy as np
from jax import lax
from jax.experimental import pallas as pl
from jax.experimental.pallas import tpu as pltpu
from jax.experimental.pallas import tpu_sc as plsc

N = 10000
NPAD = 10240
E = 160000
D = 256
DH = 128
NS = 16
NC = 2
PT = 10240
EPAD = NS * PT
CH = 80
NBLK = 8
BCH = 16
RPT = NPAD // NS
WCH = 80

_PERM = np.empty((D,), np.int32)
for _h in (0, DH):
    for _g in range(DH // 32):
        for _i in range(16):
            _PERM[_h + 32 * _g + 2 * _i] = _h + 32 * _g + _i
            _PERM[_h + 32 * _g + 2 * _i + 1] = _h + 32 * _g + 16 + _i


def _matmul_body(x_ref, w_ref, o_ref):
    o_ref[...] = jnp.dot(x_ref[...], w_ref[...],
                         preferred_element_type=jnp.float32
                         ).astype(jnp.bfloat16)


def _compute_hsplit(x, Wp):
    return pl.pallas_call(
        _matmul_body,
        grid=(10, 2),
        in_specs=[
            pl.BlockSpec((1000, D), lambda i, j: (i, 0)),
            pl.BlockSpec((D, DH), lambda i, j: (0, j)),
        ],
        out_specs=pl.BlockSpec((1000, DH), lambda i, j: (j * 10 + i, 0)),
        out_shape=jax.ShapeDtypeStruct((NC * N, DH), jnp.bfloat16),
    )(x, Wp)


_vector_mesh = plsc.VectorSubcoreMesh(core_axis_name="c", subcore_axis_name="s")

_sc_compiler_params = pltpu.CompilerParams()
if "needs_layout_passes" in pltpu.CompilerParams.__dataclass_fields__:
    _sc_compiler_params = dataclasses.replace(
        _sc_compiler_params, needs_layout_passes=False)
if "use_tc_tiling_on_sc" in pltpu.CompilerParams.__dataclass_fields__:
    _sc_compiler_params = dataclasses.replace(
        _sc_compiler_params, use_tc_tiling_on_sc=False)


@functools.partial(
    pl.kernel,
    out_type=jax.ShapeDtypeStruct((N, D), jnp.float32),
    mesh=_vector_mesh,
    compiler_params=_sc_compiler_params,
    scratch_types=[
        pltpu.VMEM_SHARED((NPAD, DH), jnp.float32),
        pltpu.VMEM((BCH, CH), jnp.int32),
        pltpu.VMEM((BCH, CH), jnp.int32),
        pltpu.VMEM((BCH, CH), jnp.int32),
        pltpu.VMEM((BCH, CH), jnp.int32),
        pltpu.VMEM((BCH, CH), jnp.float32),
        pltpu.VMEM((BCH, CH), jnp.float32),
        pltpu.VMEM((CH, DH // 2), jnp.int32),
        pltpu.VMEM((CH, DH // 2), jnp.int32),
        pltpu.VMEM((CH, DH // 2), jnp.int32),
        pltpu.VMEM((CH, DH // 2), jnp.int32),
        pltpu.VMEM((WCH, DH), jnp.float32),
        pltpu.VMEM((WCH, DH), jnp.float32),
        pltpu.SemaphoreType.DMA,
        pltpu.SemaphoreType.DMA,
        pltpu.SemaphoreType.DMA,
        pltpu.SemaphoreType.DMA,
        pltpu.SemaphoreType.DMA,
        pltpu.SemaphoreType.DMA,
        pltpu.SemaphoreType.DMA,
    ],
)
def _sc_aggregate(h_hbm, row_hbm, col_hbm, adj_hbm, z_hbm, out_hbm,
                  accum, rowA, rowB, colA, colB, valA, valB,
                  gbuf0, gbuf1, gbuf2, gbuf3, msg0, msg1,
                  gsem0, gsem1, gsem2, gsem3, ssem0, ssem1, zsem):
    c = lax.axis_index("c")
    s = lax.axis_index("s")
    r0 = s * RPT
    gbufs = (gbuf0, gbuf1, gbuf2, gbuf3)
    gsems = (gsem0, gsem1, gsem2, gsem3)
    msgs = (msg0, msg1)
    ssems = (ssem0, ssem1)
    rows = (rowA, rowB)
    cols = (colA, colB)
    vals = (valA, valB)
    NPAIR = NBLK // 2


    def load_idx(b, p):
        pltpu.sync_copy(row_hbm.at[s, b], rows[p])
        pltpu.sync_copy(col_hbm.at[c, s, b], cols[p])
        pltpu.sync_copy(adj_hbm.at[s, b], vals[p])

    def scale(gbuf, msg, val, j):
        @plsc.parallel_loop(0, CH, unroll=2)
        def _row(r):
            v = plsc.load_gather(
                val,
                [jnp.full((16,), j, jnp.int32),
                 jnp.full((16,), r, jnp.int32)])
            for g in range(DH // 32):
                w = gbuf[r, pl.ds(16 * g, 16)]
                ab = plsc.bitcast(w, jnp.bfloat16)
                va, vb = plsc.unpack(ab, format=plsc.PackFormat.INTERLEAVED)
                msg[r, pl.ds(32 * g, 16)] = va * v
                msg[r, pl.ds(32 * g + 16, 16)] = vb * v

    def wait_scatter(p, jj, m):
        pltpu.make_async_copy(msgs[m], accum.at[rows[p].at[jj]],
                              ssems[m]).wait()

    zcopy = pltpu.async_copy(z_hbm.at[pl.ds(r0, RPT)],
                             accum.at[pl.ds(r0, RPT)], zsem)
    load_idx(0, 0)
    for i in range(4):
        pltpu.async_copy(h_hbm.at[colA.at[i]], gbufs[i], gsems[i])
    zcopy.wait()
    plsc.subcore_barrier()

    @pl.loop(0, NPAIR)
    def _pair(t):
        for p in range(2):
            for jj in range(BCH):
                i = jj % 4
                m = jj % 2
                if jj == BCH - 4:
                    if p == 0:
                        load_idx(2 * t + 1, 1)
                    else:
                        @pl.when(t < NPAIR - 1)
                        def _():
                            load_idx(2 * t + 2, 0)

                pltpu.make_async_copy(h_hbm.at[cols[p].at[jj]], gbufs[i],
                                      gsems[i]).wait()

                if jj >= 2:
                    wait_scatter(p, jj - 2, m)
                elif p == 1:
                    wait_scatter(0, jj + BCH - 2, m)
                else:
                    @pl.when(t > 0)
                    def _():
                        wait_scatter(1, jj + BCH - 2, m)

                scale(gbufs[i], msgs[m], vals[p], jj)
                pltpu.async_copy(msgs[m], accum.at[rows[p].at[jj]],
                                 ssems[m], add=True)

                if jj < BCH - 4:
                    pltpu.async_copy(h_hbm.at[cols[p].at[jj + 4]],
                                     gbufs[i], gsems[i])
                elif p == 0:
                    pltpu.async_copy(h_hbm.at[colB.at[jj - (BCH - 4)]],
                                     gbufs[i], gsems[i])
                else:
                    @pl.when(t < NPAIR - 1)
                    def _():
                        pltpu.async_copy(h_hbm.at[colA.at[jj - (BCH - 4)]],
                                         gbufs[i], gsems[i])

    wait_scatter(1, BCH - 2, 0)
    wait_scatter(1, BCH - 1, 1)

    plsc.subcore_barrier()

    for t in range(RPT // WCH):
        base = r0 + t * WCH

        @pl.when(base < N)
        def _write():
            pltpu.sync_copy(accum.at[pl.ds(base, WCH)], msg0)

            @plsc.parallel_loop(0, WCH, unroll=2)
            def _relu_row(r):
                for k in range(DH // 16):
                    sl = (r, pl.ds(k * 16, 16))
                    msg0[sl] = jnp.maximum(msg0[sl], 0.0)

            pltpu.sync_copy(
                msg0, out_hbm.at[pl.ds(base, WCH), pl.ds(c * DH, DH)])


def kernel(x, edge_index, adj_values, features_nonzero, W):
    row = edge_index[0].astype(jnp.int32)
    col = edge_index[1].astype(jnp.int32)
    pad = EPAD - E
    row_p = jnp.concatenate([row, jnp.full((pad,), N, jnp.int32)])
    col_p = jnp.concatenate([col, jnp.zeros((pad,), jnp.int32)])
    adj_p = jnp.concatenate([adj_values, jnp.zeros((pad,), jnp.float32)])
    hsplit = _compute_hsplit(x, W[:, _PERM])
    h32 = lax.bitcast_convert_type(
        hsplit.reshape(NC * N, DH // 2, 2), jnp.int32)
    row4 = row_p.reshape(NS, NBLK, BCH, CH)
    col5 = jnp.stack([col_p, col_p + N]).reshape(NC, NS, NBLK, BCH, CH)
    adj4 = adj_p.reshape(NS, NBLK, BCH, CH)
    zeros = jnp.zeros((NPAD, DH), jnp.float32)
    return _sc_aggregate(h32, row4, col5, adj4, zeros)
